# baseline (speedup 1.0000x reference)
import functools

import jax
import jax.numpy as jnp
from jax import lax
from jax.experimental import pallas as pl
from jax.experimental.pallas import tpu as pltpu
from jax.experimental.pallas import tpu_sc as plsc

_E = 8
_T = 1024
_NW = 32
_L = 16
_NEG_INF = float("-inf")


def _logits_body(x_ref, w_ref, b_ref, out_ref):
    out_ref[0] = lax.dot_general(
        w_ref[...], x_ref[...],
        (((1,), (1,)), ((), ())),
        preferred_element_type=jnp.float32,
    ) + b_ref[...]


def _route_body(logits_hbm, gates_hbm, idx_hbm, lg_v, g_v, i_v):
    wid = lax.axis_index("s") * 2 + lax.axis_index("c")
    pltpu.sync_copy(logits_hbm.at[wid], lg_v)

    lanes = lax.iota(jnp.int32, _L)

    def chunk(ci, _):
        base = ci * _L
        v = [lg_v[e, pl.ds(base, _L)] for e in range(_E)]

        m1 = v[0]
        for e in range(1, _E):
            m1 = jnp.maximum(m1, v[e])
        i1 = jnp.full((_L,), _E - 1, dtype=jnp.int32)
        for e in range(_E - 2, -1, -1):
            i1 = jnp.where(v[e] == m1, jnp.int32(e), i1)

        vm = [jnp.where(i1 == e, _NEG_INF, v[e]) for e in range(_E)]
        m2 = vm[0]
        for e in range(1, _E):
            m2 = jnp.maximum(m2, vm[e])
        i2 = jnp.full((_L,), _E - 1, dtype=jnp.int32)
        for e in range(_E - 2, -1, -1):
            i2 = jnp.where(vm[e] == m2, jnp.int32(e), i2)

        e2 = jnp.exp(m2 - m1)
        r = 1.0 / (1.0 + e2)
        g1 = r
        g2 = e2 * r

        tok = base + lanes
        gbase = tok * _E
        zero = jnp.zeros((_L,), jnp.float32)
        for e in range(_E):
            ge = jnp.where(i1 == e, g1, jnp.where(i2 == e, g2, zero))
            plsc.store_scatter(g_v, [gbase + e], ge)
        ibase = tok * 2
        plsc.store_scatter(i_v, [ibase], i1)
        plsc.store_scatter(i_v, [ibase + 1], i2)
        return 0

    lax.fori_loop(0, _T // _L, chunk, 0)

    pltpu.sync_copy(g_v, gates_hbm.at[wid])
    pltpu.sync_copy(i_v, idx_hbm.at[wid])


def _route(logits_t):
    mesh = plsc.VectorSubcoreMesh(
        core_axis_name="c", subcore_axis_name="s",
        num_cores=2, num_subcores=16)
    return pl.kernel(
        _route_body,
        out_type=[
            jax.ShapeDtypeStruct((_NW, _T * _E), jnp.float32),
            jax.ShapeDtypeStruct((_NW, _T * 2), jnp.int32),
        ],
        mesh=mesh,
        scratch_types=[
            pltpu.VMEM((_E, _T), jnp.float32),
            pltpu.VMEM((_T * _E,), jnp.float32),
            pltpu.VMEM((_T * 2,), jnp.int32),
        ],
        compiler_params=pltpu.CompilerParams(needs_layout_passes=False),
    )(logits_t)


def kernel(x, W, b):
    B, N, D = x.shape
    tokens = B * N
    grid = tokens // _T
    x2 = x.reshape(tokens, D)
    b2 = b.reshape(_E, 1)

    logits_t = pl.pallas_call(
        _logits_body,
        grid=(grid,),
        in_specs=[
            pl.BlockSpec((_T, D), lambda i: (i, 0)),
            pl.BlockSpec((_E, D), lambda i: (0, 0)),
            pl.BlockSpec((_E, 1), lambda i: (0, 0)),
        ],
        out_specs=pl.BlockSpec((1, _E, _T), lambda i: (i, 0, 0)),
        out_shape=jax.ShapeDtypeStruct((grid, _E, _T), jnp.float32),
    )(x2, W, b2)

    gates, idx = _route(logits_t)
    return (gates.reshape(B, N, _E), idx.reshape(B, N, 2))


# speedup vs baseline: 1.0802x; 1.0802x over previous
import functools

import jax
import jax.numpy as jnp
from jax import lax
from jax.experimental import pallas as pl
from jax.experimental.pallas import tpu as pltpu
from jax.experimental.pallas import tpu_sc as plsc

_E = 8
_T = 1024
_NW = 32
_L = 16
_NEG_INF = float("-inf")


def _logits_body(x_ref, w_ref, b_ref, out_ref):
    out_ref[0] = lax.dot_general(
        w_ref[...], x_ref[...],
        (((1,), (1,)), ((), ())),
        preferred_element_type=jnp.float32,
    ) + b_ref[...]


def _route_body(logits_hbm, gates_hbm, idx_hbm, lg_v, g_v, i_v):
    wid = lax.axis_index("s") * 2 + lax.axis_index("c")
    nb = 8192 // _T
    bidx = wid // nb
    n0 = (wid % nb) * _T
    pltpu.sync_copy(logits_hbm.at[wid], lg_v)

    lanes = lax.iota(jnp.int32, _L)

    def chunk(ci, _):
        base = ci * _L
        v = [lg_v[e, pl.ds(base, _L)] for e in range(_E)]

        m1 = v[0]
        for e in range(1, _E):
            m1 = jnp.maximum(m1, v[e])
        i1 = jnp.full((_L,), _E - 1, dtype=jnp.int32)
        for e in range(_E - 2, -1, -1):
            i1 = jnp.where(v[e] == m1, jnp.int32(e), i1)

        vm = [jnp.where(i1 == e, _NEG_INF, v[e]) for e in range(_E)]
        m2 = vm[0]
        for e in range(1, _E):
            m2 = jnp.maximum(m2, vm[e])
        i2 = jnp.full((_L,), _E - 1, dtype=jnp.int32)
        for e in range(_E - 2, -1, -1):
            i2 = jnp.where(vm[e] == m2, jnp.int32(e), i2)

        e2 = jnp.exp(m2 - m1)
        r = 1.0 / (1.0 + e2)
        g1 = r
        g2 = e2 * r

        tok = base + lanes
        zero = jnp.zeros((_L,), jnp.float32)
        for e in range(_E):
            ge = jnp.where(i1 == e, g1, jnp.where(i2 == e, g2, zero))
            plsc.store_scatter(g_v, [tok, jnp.full((_L,), e, jnp.int32)], ge)
        plsc.store_scatter(i_v, [tok, jnp.zeros((_L,), jnp.int32)], i1)
        plsc.store_scatter(i_v, [tok, jnp.ones((_L,), jnp.int32)], i2)
        return 0

    lax.fori_loop(0, _T // _L, chunk, 0)

    pltpu.sync_copy(g_v, gates_hbm.at[bidx, pl.ds(n0, _T)])
    pltpu.sync_copy(i_v, idx_hbm.at[bidx, pl.ds(n0, _T)])


def _route(logits_t):
    mesh = plsc.VectorSubcoreMesh(
        core_axis_name="c", subcore_axis_name="s",
        num_cores=2, num_subcores=16)
    return pl.kernel(
        _route_body,
        out_type=[
            jax.ShapeDtypeStruct((4, 8192, _E), jnp.float32),
            jax.ShapeDtypeStruct((4, 8192, 2), jnp.int32),
        ],
        mesh=mesh,
        scratch_types=[
            pltpu.VMEM((_E, _T), jnp.float32),
            pltpu.VMEM((_T, _E), jnp.float32),
            pltpu.VMEM((_T, 2), jnp.int32),
        ],
        compiler_params=pltpu.CompilerParams(
            needs_layout_passes=False, use_tc_tiling_on_sc=False),
    )(logits_t)


def kernel(x, W, b):
    B, N, D = x.shape
    tokens = B * N
    grid = tokens // _T
    x2 = x.reshape(tokens, D)
    b2 = b.reshape(_E, 1)

    logits_t = pl.pallas_call(
        _logits_body,
        grid=(grid,),
        in_specs=[
            pl.BlockSpec((_T, D), lambda i: (i, 0)),
            pl.BlockSpec((_E, D), lambda i: (0, 0)),
            pl.BlockSpec((_E, 1), lambda i: (0, 0)),
        ],
        out_specs=pl.BlockSpec((1, _E, _T), lambda i: (i, 0, 0)),
        out_shape=jax.ShapeDtypeStruct((grid, _E, _T), jnp.float32),
    )(x2, W, b2)

    gates, idx = _route(logits_t)
    return (gates, idx)


# speedup vs baseline: 3.2311x; 2.9912x over previous
import jax
import jax.numpy as jnp
from jax import lax
from jax.experimental import pallas as pl

_E = 8
_T = 2048
_NEG_INF = float("-inf")


def _fused_body(x_ref, w_ref, b_ref, gates_ref, idx_ref):
    logits = lax.dot_general(
        w_ref[...], x_ref[...],
        (((1,), (1,)), ((), ())),
        preferred_element_type=jnp.float32,
    ) + b_ref[...]

    m1 = jnp.max(logits, axis=0, keepdims=True)
    i1 = jnp.full((1, _T), _E - 1, dtype=jnp.int32)
    for e in range(_E - 2, -1, -1):
        i1 = jnp.where(logits[e:e + 1, :] == m1, jnp.int32(e), i1)

    eiota = lax.broadcasted_iota(jnp.int32, (_E, _T), 0)
    masked = jnp.where(eiota == i1, _NEG_INF, logits)
    m2 = jnp.max(masked, axis=0, keepdims=True)
    i2 = jnp.full((1, _T), _E - 1, dtype=jnp.int32)
    for e in range(_E - 2, -1, -1):
        i2 = jnp.where(masked[e:e + 1, :] == m2, jnp.int32(e), i2)

    e2 = jnp.exp(m2 - m1)
    r = 1.0 / (1.0 + e2)
    g1 = r
    g2 = e2 * r

    gates_ref[0] = jnp.where(eiota == i1, g1, jnp.where(eiota == i2, g2, 0.0))
    idx_ref[0] = jnp.concatenate([i1, i2], axis=0)


def kernel(x, W, b):
    B, N, D = x.shape
    tokens = B * N
    grid = tokens // _T
    x2 = x.reshape(tokens, D)
    b2 = b.reshape(_E, 1)

    gates_t, idx_t = pl.pallas_call(
        _fused_body,
        grid=(grid,),
        in_specs=[
            pl.BlockSpec((_T, D), lambda i: (i, 0)),
            pl.BlockSpec((_E, D), lambda i: (0, 0)),
            pl.BlockSpec((_E, 1), lambda i: (0, 0)),
        ],
        out_specs=[
            pl.BlockSpec((1, _E, _T), lambda i: (i, 0, 0)),
            pl.BlockSpec((1, 2, _T), lambda i: (i, 0, 0)),
        ],
        out_shape=[
            jax.ShapeDtypeStruct((grid, _E, _T), jnp.float32),
            jax.ShapeDtypeStruct((grid, 2, _T), jnp.int32),
        ],
    )(x2, W, b2)

    full_gates = gates_t.transpose(0, 2, 1).reshape(B, N, _E)
    topk_idx = idx_t.transpose(0, 2, 1).reshape(B, N, 2)
    return (full_gates, topk_idx)


# speedup vs baseline: 3.2968x; 1.0203x over previous
import jax
import jax.numpy as jnp
from jax import lax
from jax.experimental import pallas as pl

_E = 8
_T = 4096
_NEG_INF = float("-inf")


def _fused_body(x_ref, w_ref, b_ref, gates_ref, idx_ref):
    logits = lax.dot_general(
        w_ref[...], x_ref[...],
        (((1,), (1,)), ((), ())),
        preferred_element_type=jnp.float32,
    ) + b_ref[...]

    m1 = jnp.max(logits, axis=0, keepdims=True)
    i1 = jnp.full((1, _T), _E - 1, dtype=jnp.int32)
    for e in range(_E - 2, -1, -1):
        i1 = jnp.where(logits[e:e + 1, :] == m1, jnp.int32(e), i1)

    eiota = lax.broadcasted_iota(jnp.int32, (_E, _T), 0)
    masked = jnp.where(eiota == i1, _NEG_INF, logits)
    m2 = jnp.max(masked, axis=0, keepdims=True)
    i2 = jnp.full((1, _T), _E - 1, dtype=jnp.int32)
    for e in range(_E - 2, -1, -1):
        i2 = jnp.where(masked[e:e + 1, :] == m2, jnp.int32(e), i2)

    e2 = jnp.exp(m2 - m1)
    r = 1.0 / (1.0 + e2)
    g1 = r
    g2 = e2 * r

    gates_ref[0] = jnp.where(eiota == i1, g1, jnp.where(eiota == i2, g2, 0.0))
    idx_ref[0] = jnp.concatenate([i1, i2], axis=0)


def kernel(x, W, b):
    B, N, D = x.shape
    tokens = B * N
    grid = tokens // _T
    x2 = x.reshape(tokens, D)
    b2 = b.reshape(_E, 1)

    gates_t, idx_t = pl.pallas_call(
        _fused_body,
        grid=(grid,),
        in_specs=[
            pl.BlockSpec((_T, D), lambda i: (i, 0)),
            pl.BlockSpec((_E, D), lambda i: (0, 0)),
            pl.BlockSpec((_E, 1), lambda i: (0, 0)),
        ],
        out_specs=[
            pl.BlockSpec((1, _E, _T), lambda i: (i, 0, 0)),
            pl.BlockSpec((1, 2, _T), lambda i: (i, 0, 0)),
        ],
        out_shape=[
            jax.ShapeDtypeStruct((grid, _E, _T), jnp.float32),
            jax.ShapeDtypeStruct((grid, 2, _T), jnp.int32),
        ],
    )(x2, W, b2)

    full_gates = gates_t.transpose(0, 2, 1).reshape(B, N, _E)
    topk_idx = idx_t.transpose(0, 2, 1).reshape(B, N, 2)
    return (full_gates, topk_idx)
